# baseline (speedup 1.0000x reference)
import jax
import jax.numpy as jnp
from jax.experimental import pallas as pl
from jax.experimental.pallas import tpu as pltpu
from jax.experimental.pallas import tpu_sc as plsc

B, S, H = 1, 2048, 3072
E, TOPK, FF = 8, 2, 1536
ALPHA = 1.702
LIMIT = 7.0
EPS = 1e-5
T = B * S
TT = 256
NT = T // TT
G = 24
SL = G * TT
C2 = 4
CH2 = (2 * FF) // C2
FC2 = CH2 // 2
W = 128
HC = 384
NHC = H // HC
HH = H // 2


def _router_kernel(x_ref, rmsw_ref, rwT_ref, rb_ref,
                   tnorm_ref, scores_ref, probs_ref, idx_ref):
    x = x_ref[...]
    var = jnp.mean(x * x, axis=-1, keepdims=True)
    normed = x * jax.lax.rsqrt(var + EPS) * rmsw_ref[...]
    logits = jnp.dot(normed, rwT_ref[...],
                     preferred_element_type=jnp.float32) + rb_ref[...]
    eidx = jax.lax.broadcasted_iota(jnp.int32, (TT, E), 1)
    m1 = jnp.max(logits, axis=-1, keepdims=True)
    i1 = jnp.argmax(logits, axis=-1).reshape(TT, 1)
    masked = jnp.where(eidx == i1, -jnp.inf, logits)
    m2 = jnp.max(masked, axis=-1, keepdims=True)
    i2 = jnp.argmax(masked, axis=-1).reshape(TT, 1)
    p1 = jax.nn.sigmoid(m1 - m2)
    p2 = 1.0 - p1
    scores_ref[...] = (jnp.where(eidx == i1, p1, 0.0)
                       + jnp.where(eidx == i2, p2, 0.0))
    tnorm_ref[...] = normed
    probs_ref[...] = jnp.concatenate([p1, p2], axis=-1)
    idx_ref[...] = jnp.concatenate([i1, i2], axis=-1).astype(jnp.int32)


def _meta_kernel(eT_ref, sl_ref, te_ref, tv_ref):
    f32, i32 = jnp.float32, jnp.int32
    e0 = eT_ref[0:1, :]
    e1 = eT_ref[1:2, :]
    se = jax.lax.broadcasted_iota(i32, (E, T), 0)
    m0 = (se == e0).astype(f32)
    m1 = (se == e1).astype(f32)
    colsum = m0 + m1
    ui = jax.lax.broadcasted_iota(i32, (T, T), 0)
    uj = jax.lax.broadcasted_iota(i32, (T, T), 1)
    upper = (ui < uj).astype(jnp.bfloat16)
    excl = jnp.dot(colsum.astype(jnp.bfloat16), upper,
                   preferred_element_type=f32)
    rank0 = jnp.sum(excl * m0, axis=0, keepdims=True)
    rank1 = jnp.sum(excl * m1, axis=0, keepdims=True)
    counts = excl[:, T - 1:T] + colsum[:, T - 1:T]
    ntiles = jnp.ceil(counts / TT)
    li = jax.lax.broadcasted_iota(i32, (E, E), 0)
    lj = jax.lax.broadcasted_iota(i32, (E, E), 1)
    lower = (lj < li).astype(f32)
    ft = jnp.dot(lower, ntiles, preferred_element_type=f32)
    base = ft * TT
    slot0 = jnp.sum(base * m0, axis=0, keepdims=True) + rank0
    slot1 = jnp.sum(base * m1, axis=0, keepdims=True) + rank1
    sl_ref[0:1, :] = slot0.astype(i32)
    sl_ref[1:2, :] = slot1.astype(i32)
    total = jnp.sum(ntiles)
    gi = jax.lax.broadcasted_iota(i32, (E, G), 1).astype(f32)
    te_raw = jnp.sum((ft <= gi).astype(f32), axis=0, keepdims=True) - 1.0
    gvec = jax.lax.broadcasted_iota(i32, (1, G), 1).astype(f32)
    tvv = gvec < total
    te_last = jnp.sum(te_raw * (gvec == total - 1.0), axis=1, keepdims=True)
    te_ref[...] = jnp.where(tvv, te_raw, te_last).astype(i32)
    tv_ref[...] = tvv.astype(i32)


def _sc_scatter(tnorm, sl):
    mesh = plsc.VectorSubcoreMesh(core_axis_name="c", subcore_axis_name="s")

    @pl.kernel(out_type=jax.ShapeDtypeStruct((SL, H), jnp.float32),
               mesh=mesh)
    def sck(t_hbm, sl_hbm, o_hbm):
        for h in range(NHC):
            def body(x_vmem, i_vmem, h=h):
                pltpu.sync_copy(
                    x_vmem, o_hbm.at[i_vmem.at[0], pl.ds(h * HC, HC)])

            pltpu.emit_pipeline(
                body,
                grid=(TOPK, T // W),
                in_specs=[
                    pl.BlockSpec((W, HC), index_map=lambda k, i, h=h: (i, h)),
                    pl.BlockSpec((1, W), index_map=lambda k, i: (k, i)),
                ],
                out_specs=[],
                core_axis_name=("c", "s"),
                dimension_semantics=(pltpu.PARALLEL, pltpu.PARALLEL),
            )(t_hbm, sl_hbm)

    return sck(tnorm, sl)


def _sc_gather(y, sl):
    mesh = plsc.VectorSubcoreMesh(core_axis_name="c", subcore_axis_name="s")

    @pl.kernel(out_type=jax.ShapeDtypeStruct((TOPK * T, H), jnp.float32),
               mesh=mesh)
    def gck(y_hbm, sl_hbm, o_hbm):
        for h in range(NHC):
            def body(i_vmem, o_vmem, h=h):
                pltpu.sync_copy(
                    y_hbm.at[i_vmem.at[0], pl.ds(h * HC, HC)], o_vmem)

            pltpu.emit_pipeline(
                body,
                grid=(TOPK, T // W),
                in_specs=[
                    pl.BlockSpec((1, W), index_map=lambda k, i: (k, i)),
                ],
                out_specs=[
                    pl.BlockSpec(
                        (W, HC),
                        index_map=lambda k, i, h=h: (k * (T // W) + i, h)),
                ],
                core_axis_name=("c", "s"),
                dimension_semantics=(pltpu.PARALLEL, pltpu.PARALLEL),
            )(sl_hbm, o_hbm)

    return gck(y, sl)


def _gateup_kernel(te_ref, tv_ref, x_ref, gup_ref, gub_ref,
                   act_ref, wgu_ref, pe_ref):
    f32, bf16 = jnp.float32, jnp.bfloat16
    c = pl.program_id(0)
    g = pl.program_id(1)

    changed = jnp.logical_or(
        g == 0, te_ref[jnp.maximum(g - 1, 0)] != te_ref[g])

    @pl.when(jnp.logical_and(tv_ref[g] == 1, changed))
    def _():
        wgu_ref[...] = gup_ref[0].astype(bf16)

    @pl.when(jnp.logical_and(g == 0, c == 0))
    def _():
        j = jax.lax.broadcasted_iota(jnp.int32, (CH2, FC2), 0)
        f = jax.lax.broadcasted_iota(jnp.int32, (CH2, FC2), 1)
        pe_ref[...] = (j == 2 * f).astype(bf16)

    @pl.when(tv_ref[g] == 1)
    def _():
        rows = jax.lax.broadcasted_iota(jnp.int32, (E * C2, CH2), 0)
        gub = jnp.sum(gub_ref[...] * (rows == te_ref[g] * C2 + c),
                      axis=0, keepdims=True)
        xb = x_ref[...].astype(bf16)
        gu = jnp.dot(xb, wgu_ref[...], preferred_element_type=f32) + gub
        gate = jnp.minimum(gu, LIMIT)
        glu = gate * jax.nn.sigmoid(gate * ALPHA)
        upp = jnp.clip(gu, -LIMIT, LIMIT) + 1.0
        act_i = glu * jnp.roll(upp, -1, axis=1)
        act_c = jnp.dot(act_i.astype(bf16), pe_ref[...],
                        preferred_element_type=f32)
        act_ref[...] = act_c.astype(bf16)


def _down_kernel(te_ref, tv_ref, act_ref, dw_ref, db_ref, y_ref, wdn_ref):
    f32, bf16 = jnp.float32, jnp.bfloat16
    h = pl.program_id(0)
    g = pl.program_id(1)

    changed = jnp.logical_or(
        g == 0, te_ref[jnp.maximum(g - 1, 0)] != te_ref[g])

    @pl.when(jnp.logical_and(tv_ref[g] == 1, changed))
    def _():
        wdn_ref[...] = dw_ref[0].astype(bf16)

    @pl.when(tv_ref[g] == 1)
    def _():
        rows = jax.lax.broadcasted_iota(jnp.int32, (E * 2, HH), 0)
        db = jnp.sum(db_ref[...] * (rows == te_ref[g] * 2 + h),
                     axis=0, keepdims=True)
        y_ref[...] = jnp.dot(act_ref[...], wdn_ref[...],
                             preferred_element_type=f32) + db


def _combine_kernel(z_ref, probs_ref, out_ref):
    p0 = probs_ref[:, 0:1]
    p1 = probs_ref[:, 1:2]
    out_ref[...] = p0 * z_ref[0] + p1 * z_ref[1]


def kernel(hidden_states, rms_weight, router_weight, router_bias,
           gate_up_proj, gate_up_proj_bias, down_proj, down_proj_bias):
    f32, bf16, i32 = jnp.float32, jnp.bfloat16, jnp.int32
    x = hidden_states.reshape(T, H)
    rmsw = rms_weight.reshape(1, H)
    rwT = router_weight.T
    rb = router_bias.reshape(1, E)

    tnorm, scores, probs, idx = pl.pallas_call(
        _router_kernel,
        grid=(NT,),
        in_specs=[
            pl.BlockSpec((TT, H), lambda t: (t, 0)),
            pl.BlockSpec((1, H), lambda t: (0, 0)),
            pl.BlockSpec((H, E), lambda t: (0, 0)),
            pl.BlockSpec((1, E), lambda t: (0, 0)),
        ],
        out_specs=[
            pl.BlockSpec((TT, H), lambda t: (t, 0)),
            pl.BlockSpec((TT, E), lambda t: (t, 0)),
            pl.BlockSpec((TT, 2), lambda t: (t, 0)),
            pl.BlockSpec((TT, 2), lambda t: (t, 0)),
        ],
        out_shape=[
            jax.ShapeDtypeStruct((T, H), f32),
            jax.ShapeDtypeStruct((T, E), f32),
            jax.ShapeDtypeStruct((T, 2), f32),
            jax.ShapeDtypeStruct((T, 2), i32),
        ],
    )(x, rmsw, rwT, rb)

    sl, te, tv = pl.pallas_call(
        _meta_kernel,
        grid=(1,),
        in_specs=[pl.BlockSpec((TOPK, T), lambda i: (0, 0))],
        out_specs=[
            pl.BlockSpec((TOPK, T), lambda i: (0, 0)),
            pl.BlockSpec((1, G), lambda i: (0, 0)),
            pl.BlockSpec((1, G), lambda i: (0, 0)),
        ],
        out_shape=[
            jax.ShapeDtypeStruct((TOPK, T), i32),
            jax.ShapeDtypeStruct((1, G), i32),
            jax.ShapeDtypeStruct((1, G), i32),
        ],
    )(idx.T)

    te1 = te.reshape(G)
    tv1 = tv.reshape(G)

    x_sorted = _sc_scatter(tnorm, sl)

    act = pl.pallas_call(
        _gateup_kernel,
        grid_spec=pltpu.PrefetchScalarGridSpec(
            num_scalar_prefetch=2,
            grid=(C2, G),
            in_specs=[
                pl.BlockSpec((TT, H),
                             lambda c, g, te, tv:
                             (jnp.where(tv[g] == 1, g, 0), 0)),
                pl.BlockSpec((1, H, CH2),
                             lambda c, g, te, tv: (te[g], 0, c)),
                pl.BlockSpec((E * C2, CH2), lambda c, g, te, tv: (0, 0)),
            ],
            out_specs=pl.BlockSpec((TT, FC2), lambda c, g, te, tv: (g, c)),
            scratch_shapes=[
                pltpu.VMEM((H, CH2), bf16),
                pltpu.VMEM((CH2, FC2), bf16),
            ],
        ),
        out_shape=jax.ShapeDtypeStruct((SL, C2 * FC2), bf16),
    )(te1, tv1, x_sorted, gate_up_proj,
      gate_up_proj_bias.reshape(E * C2, CH2))

    y = pl.pallas_call(
        _down_kernel,
        grid_spec=pltpu.PrefetchScalarGridSpec(
            num_scalar_prefetch=2,
            grid=(2, G),
            in_specs=[
                pl.BlockSpec((TT, FF),
                             lambda h, g, te, tv:
                             (jnp.where(tv[g] == 1, g, 0), 0)),
                pl.BlockSpec((1, FF, HH),
                             lambda h, g, te, tv: (te[g], 0, h)),
                pl.BlockSpec((E * 2, HH), lambda h, g, te, tv: (0, 0)),
            ],
            out_specs=pl.BlockSpec((TT, HH), lambda h, g, te, tv: (g, h)),
            scratch_shapes=[pltpu.VMEM((FF, HH), bf16)],
        ),
        out_shape=jax.ShapeDtypeStruct((SL, H), f32),
    )(te1, tv1, act, down_proj, down_proj_bias.reshape(E * 2, HH))

    z = _sc_gather(y, sl).reshape(TOPK, T, H)

    out = pl.pallas_call(
        _combine_kernel,
        grid=(NT,),
        in_specs=[
            pl.BlockSpec((TOPK, TT, H), lambda t: (0, t, 0)),
            pl.BlockSpec((TT, 2), lambda t: (t, 0)),
        ],
        out_specs=pl.BlockSpec((TT, H), lambda t: (t, 0)),
        out_shape=jax.ShapeDtypeStruct((T, H), f32),
    )(z, probs)

    return (out.reshape(B, S, H), scores)


# speedup vs baseline: 1.0283x; 1.0283x over previous
import jax
import jax.numpy as jnp
from jax.experimental import pallas as pl
from jax.experimental.pallas import tpu as pltpu
from jax.experimental.pallas import tpu_sc as plsc

B, S, H = 1, 2048, 3072
E, TOPK, FF = 8, 2, 1536
ALPHA = 1.702
LIMIT = 7.0
EPS = 1e-5
T = B * S
TT = 256
NT = T // TT
G = 24
SL = G * TT
C2 = 4
CH2 = (2 * FF) // C2
FC2 = CH2 // 2
W = 128
HC = 384
NHC = H // HC
HH = H // 2


def _router_kernel(x_ref, rmsw_ref, rwT_ref, rb_ref,
                   tnorm_ref, scores_ref, probs_ref, idx_ref):
    x = x_ref[...]
    var = jnp.mean(x * x, axis=-1, keepdims=True)
    normed = x * jax.lax.rsqrt(var + EPS) * rmsw_ref[...]
    logits = jnp.dot(normed, rwT_ref[...],
                     preferred_element_type=jnp.float32) + rb_ref[...]
    eidx = jax.lax.broadcasted_iota(jnp.int32, (TT, E), 1)
    m1 = jnp.max(logits, axis=-1, keepdims=True)
    i1 = jnp.argmax(logits, axis=-1).reshape(TT, 1)
    masked = jnp.where(eidx == i1, -jnp.inf, logits)
    m2 = jnp.max(masked, axis=-1, keepdims=True)
    i2 = jnp.argmax(masked, axis=-1).reshape(TT, 1)
    p1 = jax.nn.sigmoid(m1 - m2)
    p2 = 1.0 - p1
    scores_ref[...] = (jnp.where(eidx == i1, p1, 0.0)
                       + jnp.where(eidx == i2, p2, 0.0))
    tnorm_ref[...] = normed
    probs_ref[...] = jnp.concatenate([p1, p2], axis=-1)
    idx_ref[...] = jnp.concatenate([i1, i2], axis=-1).astype(jnp.int32)


def _meta_kernel(eT_ref, sl_ref, te_ref, tv_ref):
    f32, i32 = jnp.float32, jnp.int32
    e0 = eT_ref[0:1, :]
    e1 = eT_ref[1:2, :]
    se = jax.lax.broadcasted_iota(i32, (E, T), 0)
    m0 = (se == e0).astype(f32)
    m1 = (se == e1).astype(f32)
    colsum = m0 + m1
    ui = jax.lax.broadcasted_iota(i32, (T, T), 0)
    uj = jax.lax.broadcasted_iota(i32, (T, T), 1)
    upper = (ui < uj).astype(jnp.bfloat16)
    excl = jnp.dot(colsum.astype(jnp.bfloat16), upper,
                   preferred_element_type=f32)
    rank0 = jnp.sum(excl * m0, axis=0, keepdims=True)
    rank1 = jnp.sum(excl * m1, axis=0, keepdims=True)
    counts = excl[:, T - 1:T] + colsum[:, T - 1:T]
    ntiles = jnp.ceil(counts / TT)
    li = jax.lax.broadcasted_iota(i32, (E, E), 0)
    lj = jax.lax.broadcasted_iota(i32, (E, E), 1)
    lower = (lj < li).astype(f32)
    ft = jnp.dot(lower, ntiles, preferred_element_type=f32)
    base = ft * TT
    slot0 = jnp.sum(base * m0, axis=0, keepdims=True) + rank0
    slot1 = jnp.sum(base * m1, axis=0, keepdims=True) + rank1
    sl_ref[0:1, :] = slot0.astype(i32)
    sl_ref[1:2, :] = slot1.astype(i32)
    total = jnp.sum(ntiles)
    gi = jax.lax.broadcasted_iota(i32, (E, G), 1).astype(f32)
    te_raw = jnp.sum((ft <= gi).astype(f32), axis=0, keepdims=True) - 1.0
    gvec = jax.lax.broadcasted_iota(i32, (1, G), 1).astype(f32)
    tvv = gvec < total
    te_last = jnp.sum(te_raw * (gvec == total - 1.0), axis=1, keepdims=True)
    te_ref[...] = jnp.where(tvv, te_raw, te_last).astype(i32)
    tv_ref[...] = tvv.astype(i32)


def _sc_scatter(tnorm, sl):
    mesh = plsc.VectorSubcoreMesh(core_axis_name="c", subcore_axis_name="s")

    @pl.kernel(out_type=jax.ShapeDtypeStruct((SL, H), jnp.float32),
               mesh=mesh)
    def sck(t_hbm, sl_hbm, o_hbm):
        for h in range(NHC):
            def body(x_vmem, i_vmem, h=h):
                pltpu.sync_copy(
                    x_vmem, o_hbm.at[i_vmem.at[0], pl.ds(h * HC, HC)])

            pltpu.emit_pipeline(
                body,
                grid=(TOPK, T // W),
                in_specs=[
                    pl.BlockSpec((W, HC), index_map=lambda k, i, h=h: (i, h)),
                    pl.BlockSpec((1, W), index_map=lambda k, i: (k, i)),
                ],
                out_specs=[],
                core_axis_name=("c", "s"),
                dimension_semantics=(pltpu.PARALLEL, pltpu.PARALLEL),
            )(t_hbm, sl_hbm)

    return sck(tnorm, sl)


def _sc_gather(y, sl):
    mesh = plsc.VectorSubcoreMesh(core_axis_name="c", subcore_axis_name="s")

    @pl.kernel(out_type=jax.ShapeDtypeStruct((TOPK * T, H), jnp.float32),
               mesh=mesh)
    def gck(y_hbm, sl_hbm, o_hbm):
        for h in range(NHC):
            def body(i_vmem, o_vmem, h=h):
                pltpu.sync_copy(
                    y_hbm.at[i_vmem.at[0], pl.ds(h * HC, HC)], o_vmem)

            pltpu.emit_pipeline(
                body,
                grid=(TOPK, T // W),
                in_specs=[
                    pl.BlockSpec((1, W), index_map=lambda k, i: (k, i)),
                ],
                out_specs=[
                    pl.BlockSpec(
                        (W, HC),
                        index_map=lambda k, i, h=h: (k * (T // W) + i, h)),
                ],
                core_axis_name=("c", "s"),
                dimension_semantics=(pltpu.PARALLEL, pltpu.PARALLEL),
            )(sl_hbm, o_hbm)

    return gck(y, sl)


def _gateup_kernel(te_ref, tv_ref, x_ref, gup_ref, gub_ref,
                   act_ref, wgu_ref, pe_ref):
    f32, bf16 = jnp.float32, jnp.bfloat16
    c = pl.program_id(0)
    g = pl.program_id(1)

    @pl.when(tv_ref[g] == 1)
    def _():
        wgu_ref[...] = gup_ref[0].astype(bf16)
        j = jax.lax.broadcasted_iota(jnp.int32, (CH2, FC2), 0)
        f = jax.lax.broadcasted_iota(jnp.int32, (CH2, FC2), 1)
        pe_ref[...] = (j == 2 * f).astype(bf16)
        rows = jax.lax.broadcasted_iota(jnp.int32, (E * C2, CH2), 0)
        gub = jnp.sum(gub_ref[...] * (rows == te_ref[g] * C2 + c),
                      axis=0, keepdims=True)
        xb = x_ref[...].astype(bf16)
        gu = jnp.dot(xb, wgu_ref[...], preferred_element_type=f32) + gub
        gate = jnp.minimum(gu, LIMIT)
        glu = gate * jax.nn.sigmoid(gate * ALPHA)
        upp = jnp.clip(gu, -LIMIT, LIMIT) + 1.0
        act_i = glu * jnp.roll(upp, -1, axis=1)
        act_c = jnp.dot(act_i.astype(bf16), pe_ref[...],
                        preferred_element_type=f32)
        act_ref[...] = act_c.astype(bf16)


def _down_kernel(te_ref, tv_ref, act_ref, dw_ref, db_ref, y_ref, wdn_ref):
    f32, bf16 = jnp.float32, jnp.bfloat16
    h = pl.program_id(0)
    g = pl.program_id(1)

    @pl.when(tv_ref[g] == 1)
    def _():
        wdn_ref[...] = dw_ref[0].astype(bf16)
        rows = jax.lax.broadcasted_iota(jnp.int32, (E * 2, HH), 0)
        db = jnp.sum(db_ref[...] * (rows == te_ref[g] * 2 + h),
                     axis=0, keepdims=True)
        y_ref[...] = jnp.dot(act_ref[...], wdn_ref[...],
                             preferred_element_type=f32) + db


def _combine_kernel(z_ref, probs_ref, out_ref):
    p0 = probs_ref[:, 0:1]
    p1 = probs_ref[:, 1:2]
    out_ref[...] = p0 * z_ref[0] + p1 * z_ref[1]


def kernel(hidden_states, rms_weight, router_weight, router_bias,
           gate_up_proj, gate_up_proj_bias, down_proj, down_proj_bias):
    f32, bf16, i32 = jnp.float32, jnp.bfloat16, jnp.int32
    x = hidden_states.reshape(T, H)
    rmsw = rms_weight.reshape(1, H)
    rwT = router_weight.T
    rb = router_bias.reshape(1, E)

    tnorm, scores, probs, idx = pl.pallas_call(
        _router_kernel,
        grid=(NT,),
        in_specs=[
            pl.BlockSpec((TT, H), lambda t: (t, 0)),
            pl.BlockSpec((1, H), lambda t: (0, 0)),
            pl.BlockSpec((H, E), lambda t: (0, 0)),
            pl.BlockSpec((1, E), lambda t: (0, 0)),
        ],
        out_specs=[
            pl.BlockSpec((TT, H), lambda t: (t, 0)),
            pl.BlockSpec((TT, E), lambda t: (t, 0)),
            pl.BlockSpec((TT, 2), lambda t: (t, 0)),
            pl.BlockSpec((TT, 2), lambda t: (t, 0)),
        ],
        out_shape=[
            jax.ShapeDtypeStruct((T, H), f32),
            jax.ShapeDtypeStruct((T, E), f32),
            jax.ShapeDtypeStruct((T, 2), f32),
            jax.ShapeDtypeStruct((T, 2), i32),
        ],
    )(x, rmsw, rwT, rb)

    sl, te, tv = pl.pallas_call(
        _meta_kernel,
        grid=(1,),
        in_specs=[pl.BlockSpec((TOPK, T), lambda i: (0, 0))],
        out_specs=[
            pl.BlockSpec((TOPK, T), lambda i: (0, 0)),
            pl.BlockSpec((1, G), lambda i: (0, 0)),
            pl.BlockSpec((1, G), lambda i: (0, 0)),
        ],
        out_shape=[
            jax.ShapeDtypeStruct((TOPK, T), i32),
            jax.ShapeDtypeStruct((1, G), i32),
            jax.ShapeDtypeStruct((1, G), i32),
        ],
    )(idx.T)

    te1 = te.reshape(G)
    tv1 = tv.reshape(G)

    x_sorted = _sc_scatter(tnorm, sl)

    act = pl.pallas_call(
        _gateup_kernel,
        grid_spec=pltpu.PrefetchScalarGridSpec(
            num_scalar_prefetch=2,
            grid=(C2, G),
            in_specs=[
                pl.BlockSpec((TT, H),
                             lambda c, g, te, tv:
                             (jnp.where(tv[g] == 1, g, 0), 0)),
                pl.BlockSpec((1, H, CH2),
                             lambda c, g, te, tv: (te[g], 0, c)),
                pl.BlockSpec((E * C2, CH2), lambda c, g, te, tv: (0, 0)),
            ],
            out_specs=pl.BlockSpec((TT, FC2), lambda c, g, te, tv: (g, c)),
            scratch_shapes=[
                pltpu.VMEM((H, CH2), bf16),
                pltpu.VMEM((CH2, FC2), bf16),
            ],
        ),
        out_shape=jax.ShapeDtypeStruct((SL, C2 * FC2), bf16),
    )(te1, tv1, x_sorted, gate_up_proj,
      gate_up_proj_bias.reshape(E * C2, CH2))

    y = pl.pallas_call(
        _down_kernel,
        grid_spec=pltpu.PrefetchScalarGridSpec(
            num_scalar_prefetch=2,
            grid=(2, G),
            in_specs=[
                pl.BlockSpec((TT, FF),
                             lambda h, g, te, tv:
                             (jnp.where(tv[g] == 1, g, 0), 0)),
                pl.BlockSpec((1, FF, HH),
                             lambda h, g, te, tv: (te[g], 0, h)),
                pl.BlockSpec((E * 2, HH), lambda h, g, te, tv: (0, 0)),
            ],
            out_specs=pl.BlockSpec((TT, HH), lambda h, g, te, tv: (g, h)),
            scratch_shapes=[pltpu.VMEM((FF, HH), bf16)],
        ),
        out_shape=jax.ShapeDtypeStruct((SL, H), f32),
    )(te1, tv1, act, down_proj, down_proj_bias.reshape(E * 2, HH))

    z = _sc_gather(y, sl).reshape(TOPK, T, H)

    out = pl.pallas_call(
        _combine_kernel,
        grid=(NT,),
        in_specs=[
            pl.BlockSpec((TOPK, TT, H), lambda t: (0, t, 0)),
            pl.BlockSpec((TT, 2), lambda t: (t, 0)),
        ],
        out_specs=pl.BlockSpec((TT, H), lambda t: (t, 0)),
        out_shape=jax.ShapeDtypeStruct((T, H), f32),
    )(z, probs)

    return (out.reshape(B, S, H), scores)
